# Initial kernel scaffold; baseline (speedup 1.0000x reference)
import jax
import jax.numpy as jnp
from jax.experimental import pallas as pl


def kernel(x, edge_index, edge_label_index, W1_l, W1_r, b1, W2_l, W2_r, b2, W3_l, W3_r, b3):
    raise NotImplementedError("write your pallas kernel here")


# speedup vs baseline: 5.9940x; 5.9940x over previous
import functools

import jax
import jax.numpy as jnp
from jax import lax
from jax.experimental import pallas as pl
from jax.experimental.pallas import tpu as pltpu
from jax.experimental.pallas import tpu_sc as plsc

F32 = jnp.float32
I32 = jnp.int32

NC = 2
NS = 16
NW = NC * NS
CHUNK = 128
WID = 128
D_H = 64
N_REAL = 10000
N_TAB = 10240
RPT = N_TAB // NS


def _mesh():
    return plsc.VectorSubcoreMesh(core_axis_name="c", subcore_axis_name="s")


_SC_PARAMS = pltpu.CompilerParams(needs_layout_passes=False)


def _make_agg(n_chunk_rows):
    cpw = n_chunk_rows // NW

    def body(g_hbm, src_hbm, dst_hbm, zero_hbm, acc_out,
             idx_s, idx_d, rows, acc_sh, sem):
        c = lax.axis_index("c")
        s = lax.axis_index("s")
        wid = c * NS + s
        r0 = s * RPT
        pltpu.sync_copy(zero_hbm.at[pl.ds(r0, RPT)], acc_sh.at[pl.ds(r0, RPT)])
        plsc.subcore_barrier()

        def chunk(i, carry):
            row = wid * cpw + i
            pltpu.sync_copy(src_hbm.at[row], idx_s)
            pltpu.sync_copy(dst_hbm.at[row], idx_d)
            pltpu.async_copy(g_hbm.at[idx_s], rows, sem).wait()
            pltpu.sync_copy(rows, acc_sh.at[idx_d], add=True)
            return carry

        lax.fori_loop(0, cpw, chunk, 0)
        plsc.subcore_barrier()
        pltpu.sync_copy(acc_sh.at[pl.ds(r0, RPT)],
                        acc_out.at[c, pl.ds(r0, RPT)])

    return pl.kernel(
        body,
        out_type=jax.ShapeDtypeStruct((NC, N_TAB, WID), F32),
        mesh=_mesh(),
        compiler_params=_SC_PARAMS,
        scratch_types=(
            pltpu.VMEM((CHUNK,), I32),
            pltpu.VMEM((CHUNK,), I32),
            pltpu.VMEM((CHUNK, WID), F32),
            pltpu.VMEM_SHARED((N_TAB, WID), F32),
            pltpu.SemaphoreType.DMA,
        ),
    )


def _make_pred(n_chunk_rows):
    cpw = n_chunk_rows // NW

    def body(h_hbm, s_hbm, d_hbm, out_hbm,
             sidx, didx, srows, drows, ovec, pbuf, h_sh, sem):
        c = lax.axis_index("c")
        s = lax.axis_index("s")
        wid = c * NS + s
        r0 = s * RPT
        pltpu.sync_copy(h_hbm.at[pl.ds(r0, RPT)], h_sh.at[pl.ds(r0, RPT)])
        plsc.subcore_barrier()
        lanes = jnp.arange(16, dtype=I32)

        def chunk(i, carry):
            row = wid * cpw + i
            pltpu.sync_copy(s_hbm.at[row], sidx)
            pltpu.sync_copy(d_hbm.at[row], didx)
            pltpu.async_copy(h_sh.at[sidx], srows, sem).wait()
            pltpu.async_copy(h_sh.at[didx], drows, sem).wait()

            def grp(g, carry2):
                def rowp(r, carry3):
                    rr = g * 16 + r
                    p = jnp.zeros((16,), F32)
                    for k in range(D_H // 16):
                        sv = srows[rr, pl.ds(k * 16, 16)]
                        dv = drows[rr, pl.ds(k * 16, 16)]
                        p = p + sv * dv
                    plsc.store_scatter(pbuf, [r * 16 + lanes], p)
                    return carry3

                lax.fori_loop(0, 16, rowp, 0)
                acc = jnp.zeros((16,), F32)
                for l in range(16):
                    acc = acc + plsc.load_gather(pbuf, [lanes * 16 + l])
                plsc.store_scatter(ovec, [g * 16 + lanes], acc)
                return carry2

            lax.fori_loop(0, CHUNK // 16, grp, 0)
            pltpu.sync_copy(ovec, out_hbm.at[row])
            return carry

        lax.fori_loop(0, cpw, chunk, 0)

    return pl.kernel(
        body,
        out_type=jax.ShapeDtypeStruct((n_chunk_rows, CHUNK), F32),
        mesh=_mesh(),
        compiler_params=_SC_PARAMS,
        scratch_types=(
            pltpu.VMEM((CHUNK,), I32),
            pltpu.VMEM((CHUNK,), I32),
            pltpu.VMEM((CHUNK, WID), F32),
            pltpu.VMEM((CHUNK, WID), F32),
            pltpu.VMEM((CHUNK,), F32),
            pltpu.VMEM((256,), F32),
            pltpu.VMEM_SHARED((N_TAB, WID), F32),
            pltpu.SemaphoreType.DMA,
        ),
    )


M_BLK = 1280


def _pack_g(o, blk_i, g_ref, z_ref):
    rows = blk_i * M_BLK + lax.broadcasted_iota(I32, (M_BLK, 1), 0)
    mask = rows < N_REAL
    g_ref[:, :D_H] = jnp.where(mask, o[:, :D_H], 0.0)
    g_ref[:, D_H:D_H + 1] = jnp.where(mask, 1.0, 0.0)
    g_ref[:, D_H + 1:] = jnp.zeros((M_BLK, WID - D_H - 1), F32)
    z_ref[...] = o[:, D_H:]


def _dense_body(x_ref, w_ref, b_ref, g_ref, z_ref):
    o = jnp.dot(x_ref[...], w_ref[...], preferred_element_type=F32)
    o = o + b_ref[...]
    _pack_g(o, pl.program_id(0), g_ref, z_ref)


def _dense(xp, wcat, bcat):
    k = xp.shape[1]
    return pl.pallas_call(
        _dense_body,
        grid=(N_TAB // M_BLK,),
        in_specs=[
            pl.BlockSpec((M_BLK, k), lambda i: (i, 0)),
            pl.BlockSpec((k, 2 * D_H), lambda i: (0, 0)),
            pl.BlockSpec((1, 2 * D_H), lambda i: (0, 0)),
        ],
        out_specs=[
            pl.BlockSpec((M_BLK, WID), lambda i: (i, 0)),
            pl.BlockSpec((M_BLK, D_H), lambda i: (i, 0)),
        ],
        out_shape=[
            jax.ShapeDtypeStruct((N_TAB, WID), F32),
            jax.ShapeDtypeStruct((N_TAB, D_H), F32),
        ],
    )(xp, wcat, bcat)


def _agg_h(acc_ref, z_ref):
    a = acc_ref[0, :, :D_H] + acc_ref[1, :, :D_H]
    cnt = acc_ref[0, :, D_H:D_H + 1] + acc_ref[1, :, D_H:D_H + 1]
    inv = 1.0 / jnp.maximum(cnt, 1.0)
    return a * inv + z_ref[...]


def _mid_body(acc_ref, z_ref, w_ref, b_ref, g_ref, zo_ref):
    h = jnp.maximum(_agg_h(acc_ref, z_ref), 0.0)
    o = jnp.dot(h, w_ref[...], preferred_element_type=F32) + b_ref[...]
    _pack_g(o, pl.program_id(0), g_ref, zo_ref)


def _mid(acc, z, wcat, bcat):
    return pl.pallas_call(
        _mid_body,
        grid=(N_TAB // M_BLK,),
        in_specs=[
            pl.BlockSpec((NC, M_BLK, WID), lambda i: (0, i, 0)),
            pl.BlockSpec((M_BLK, D_H), lambda i: (i, 0)),
            pl.BlockSpec((D_H, 2 * D_H), lambda i: (0, 0)),
            pl.BlockSpec((1, 2 * D_H), lambda i: (0, 0)),
        ],
        out_specs=[
            pl.BlockSpec((M_BLK, WID), lambda i: (i, 0)),
            pl.BlockSpec((M_BLK, D_H), lambda i: (i, 0)),
        ],
        out_shape=[
            jax.ShapeDtypeStruct((N_TAB, WID), F32),
            jax.ShapeDtypeStruct((N_TAB, D_H), F32),
        ],
    )(acc, z, wcat, bcat)


def _fin_body(acc_ref, z_ref, h_ref):
    h_ref[:, :D_H] = _agg_h(acc_ref, z_ref)
    h_ref[:, D_H:] = jnp.zeros((M_BLK, WID - D_H), F32)


def _fin(acc, z):
    return pl.pallas_call(
        _fin_body,
        grid=(N_TAB // M_BLK,),
        in_specs=[
            pl.BlockSpec((NC, M_BLK, WID), lambda i: (0, i, 0)),
            pl.BlockSpec((M_BLK, D_H), lambda i: (i, 0)),
        ],
        out_specs=pl.BlockSpec((M_BLK, WID), lambda i: (i, 0)),
        out_shape=jax.ShapeDtypeStruct((N_TAB, WID), F32),
    )(acc, z)


def _pad_pairs(a, b):
    e = a.shape[0]
    rows = -(-e // (CHUNK * NW)) * NW
    pad = N_REAL + (jnp.arange(rows * CHUNK - e, dtype=I32) % 16)
    a2 = jnp.concatenate([a.astype(I32), pad]).reshape(rows, CHUNK)
    b2 = jnp.concatenate([b.astype(I32), pad]).reshape(rows, CHUNK)
    return a2, b2, rows


def kernel(x, edge_index, edge_label_index,
           W1_l, W1_r, b1, W2_l, W2_r, b2, W3_l, W3_r, b3):
    nl = edge_label_index.shape[1]

    xp = jnp.zeros((N_TAB, x.shape[1]), F32).at[:x.shape[0]].set(x)
    src2d, dst2d, e_rows = _pad_pairs(edge_index[0], edge_index[1])
    s2d, d2d, l_rows = _pad_pairs(edge_label_index[0], edge_label_index[1])

    zero_tab = jnp.zeros((N_TAB, WID), F32)

    def cat(wl, wr, b):
        wc = jnp.concatenate([wl, wr], axis=1)
        bc = jnp.concatenate([jnp.zeros((D_H,), F32), b]).reshape(1, 2 * D_H)
        return wc, bc

    w1c, b1c = cat(W1_l, W1_r, b1)
    w2c, b2c = cat(W2_l, W2_r, b2)
    w3c, b3c = cat(W3_l, W3_r, b3)

    agg = _make_agg(e_rows)
    pred = _make_pred(l_rows)

    g1, z1 = _dense(xp, w1c, b1c)
    acc1 = agg(g1, src2d, dst2d, zero_tab)
    g2, z2 = _mid(acc1, z1, w2c, b2c)
    acc2 = agg(g2, src2d, dst2d, zero_tab)
    g3, z3 = _mid(acc2, z2, w3c, b3c)
    acc3 = agg(g3, src2d, dst2d, zero_tab)
    h3 = _fin(acc3, z3)
    out = pred(h3, s2d, d2d)
    return out.reshape(-1)[:nl]


# speedup vs baseline: 8.2215x; 1.3716x over previous
import functools

import jax
import jax.numpy as jnp
from jax import lax
from jax.experimental import pallas as pl
from jax.experimental.pallas import tpu as pltpu
from jax.experimental.pallas import tpu_sc as plsc

F32 = jnp.float32
I32 = jnp.int32

NC = 2
NS = 16
NW = NC * NS
CHUNK = 128
WID = 128
D_H = 64
N_REAL = 10000
N_TAB = 10240
RPT = N_TAB // NS


def _mesh():
    return plsc.VectorSubcoreMesh(core_axis_name="c", subcore_axis_name="s")


_SC_PARAMS = pltpu.CompilerParams(needs_layout_passes=False)


def _make_agg(n_chunk_rows):
    cpw = n_chunk_rows // NW
    hi = cpw // 2

    def body(g_hbm, src_hbm, dst_hbm, zero_hbm, acc_out,
             sidx, dring, buf0, buf1, acc_sh, gsem, ssem, dsem):
        c = lax.axis_index("c")
        s = lax.axis_index("s")
        wid = c * NS + s
        r0 = s * RPT
        row0 = wid * cpw
        pltpu.sync_copy(zero_hbm.at[pl.ds(r0, RPT)], acc_sh.at[pl.ds(r0, RPT)])
        pltpu.sync_copy(src_hbm.at[pl.ds(row0, cpw)], sidx)
        plsc.subcore_barrier()
        bufs = (buf0, buf1)

        def g_start(j, buf):
            pltpu.async_copy(g_hbm.at[sidx.at[j]], buf, gsem)

        def g_wait(j, buf):
            pltpu.make_async_copy(g_hbm.at[sidx.at[j]], buf, gsem).wait()

        def d_start(j, u):
            pltpu.async_copy(dst_hbm.at[pl.ds((row0 + j) * CHUNK, CHUNK)],
                             dring.at[u], dsem)

        def d_wait(j, u):
            pltpu.make_async_copy(dst_hbm.at[pl.ds((row0 + j) * CHUNK, CHUNK)],
                                  dring.at[u], dsem).wait()

        def s_start(j, u, buf):
            pltpu.async_copy(buf, acc_sh.at[dring.at[u]], ssem, add=True)

        def s_wait(j, u, buf):
            pltpu.make_async_copy(buf, acc_sh.at[dring.at[u]], ssem).wait()

        for u in range(4):
            d_start(u, u)
        g_start(0, buf0)

        def io_body(io, carry):
            base = 4 * io
            for u in range(4):
                j = base + u
                g_wait(j, bufs[u % 2])
                d_wait(j, u)

                @pl.when(j > 0)
                def _(u=u, j=j):
                    s_wait(j - 1, (u - 1) % 4, bufs[(u - 1) % 2])

                @pl.when(jnp.logical_and(j > 0, j + 3 < cpw))
                def _(u=u, j=j):
                    d_start(j + 3, (u + 3) % 4)

                @pl.when(j + 1 < cpw)
                def _(u=u, j=j):
                    g_start(j + 1, bufs[(u + 1) % 2])

                s_start(j, u, bufs[u % 2])
            return carry

        lax.fori_loop(0, cpw // 4, io_body, 0)
        s_wait(cpw - 1, 3, bufs[1])
        plsc.subcore_barrier()
        pltpu.sync_copy(acc_sh.at[pl.ds(r0, RPT)],
                        acc_out.at[c, pl.ds(r0, RPT)])

    return pl.kernel(
        body,
        out_type=jax.ShapeDtypeStruct((NC, N_TAB, WID), F32),
        mesh=_mesh(),
        compiler_params=_SC_PARAMS,
        scratch_types=(
            pltpu.VMEM((cpw, CHUNK), I32),
            pltpu.VMEM((4, CHUNK), I32),
            pltpu.VMEM((CHUNK, WID), F32),
            pltpu.VMEM((CHUNK, WID), F32),
            pltpu.VMEM_SHARED((N_TAB, WID), F32),
            pltpu.SemaphoreType.DMA,
            pltpu.SemaphoreType.DMA,
            pltpu.SemaphoreType.DMA,
        ),
    )


def _make_pred(n_chunk_rows):
    cpw = n_chunk_rows // NW
    hi = cpw // 2

    def body(h_hbm, s_hbm, d_hbm, out_hbm,
             sidx, didx, sr0, dr0, sr1, dr1, ovec, pbuf, sem):
        c = lax.axis_index("c")
        s = lax.axis_index("s")
        wid = c * NS + s
        row0 = wid * cpw
        pltpu.sync_copy(s_hbm.at[pl.ds(row0, cpw)], sidx)
        pltpu.sync_copy(d_hbm.at[pl.ds(row0, cpw)], didx)
        lanes = jnp.arange(16, dtype=I32)

        def g_start(j, sbuf, dbuf):
            pltpu.async_copy(h_hbm.at[sidx.at[j]], sbuf, sem)
            pltpu.async_copy(h_hbm.at[didx.at[j]], dbuf, sem)

        def g_wait(j, sbuf, dbuf):
            pltpu.make_async_copy(h_hbm.at[sidx.at[j]], sbuf, sem).wait()
            pltpu.make_async_copy(h_hbm.at[didx.at[j]], dbuf, sem).wait()

        def compute(j, sbuf, dbuf):
            def grp(g, carry2):
                def rowp(r, carry3):
                    rr = g * 16 + r
                    p = jnp.zeros((16,), F32)
                    for k in range(D_H // 16):
                        sv = sbuf[rr, pl.ds(k * 16, 16)]
                        dv = dbuf[rr, pl.ds(k * 16, 16)]
                        p = p + sv * dv
                    plsc.store_scatter(pbuf, [r * 16 + lanes], p)
                    return carry3

                lax.fori_loop(0, 16, rowp, 0)
                acc = jnp.zeros((16,), F32)
                for l in range(16):
                    acc = acc + plsc.load_gather(pbuf, [lanes * 16 + l])
                plsc.store_scatter(ovec, [g * 16 + lanes], acc)
                return carry2

            lax.fori_loop(0, CHUNK // 16, grp, 0)
            pltpu.sync_copy(ovec,
                            out_hbm.at[pl.ds((row0 + j) * CHUNK, CHUNK)])

        g_start(0, sr0, dr0)

        def io_body(io, carry):
            j0 = 2 * io
            j1 = j0 + 1
            g_wait(j0, sr0, dr0)
            g_start(j1, sr1, dr1)
            compute(j0, sr0, dr0)
            g_wait(j1, sr1, dr1)

            @pl.when(io < hi - 1)
            def _():
                g_start(j0 + 2, sr0, dr0)

            compute(j1, sr1, dr1)
            return carry

        lax.fori_loop(0, hi, io_body, 0)

    return pl.kernel(
        body,
        out_type=jax.ShapeDtypeStruct((n_chunk_rows * CHUNK,), F32),
        mesh=_mesh(),
        compiler_params=_SC_PARAMS,
        scratch_types=(
            pltpu.VMEM((cpw, CHUNK), I32),
            pltpu.VMEM((cpw, CHUNK), I32),
            pltpu.VMEM((CHUNK, WID), F32),
            pltpu.VMEM((CHUNK, WID), F32),
            pltpu.VMEM((CHUNK, WID), F32),
            pltpu.VMEM((CHUNK, WID), F32),
            pltpu.VMEM((CHUNK,), F32),
            pltpu.VMEM((256,), F32),
            pltpu.SemaphoreType.DMA,
        ),
    )


M_BLK = 1280


def _pack_g(o, blk_i, g_ref, z_ref):
    rows = blk_i * M_BLK + lax.broadcasted_iota(I32, (M_BLK, 1), 0)
    mask = rows < N_REAL
    g_ref[:, :D_H] = jnp.where(mask, o[:, :D_H], 0.0)
    g_ref[:, D_H:D_H + 1] = jnp.where(mask, 1.0, 0.0)
    g_ref[:, D_H + 1:] = jnp.zeros((M_BLK, WID - D_H - 1), F32)
    z_ref[...] = o[:, D_H:]


def _dense_body(x_ref, w_ref, b_ref, g_ref, z_ref):
    o = jnp.dot(x_ref[...], w_ref[...], preferred_element_type=F32)
    o = o + b_ref[...]
    _pack_g(o, pl.program_id(0), g_ref, z_ref)


def _dense(xp, wcat, bcat):
    k = xp.shape[1]
    return pl.pallas_call(
        _dense_body,
        grid=(N_TAB // M_BLK,),
        in_specs=[
            pl.BlockSpec((M_BLK, k), lambda i: (i, 0)),
            pl.BlockSpec((k, 2 * D_H), lambda i: (0, 0)),
            pl.BlockSpec((1, 2 * D_H), lambda i: (0, 0)),
        ],
        out_specs=[
            pl.BlockSpec((M_BLK, WID), lambda i: (i, 0)),
            pl.BlockSpec((M_BLK, D_H), lambda i: (i, 0)),
        ],
        out_shape=[
            jax.ShapeDtypeStruct((N_TAB, WID), F32),
            jax.ShapeDtypeStruct((N_TAB, D_H), F32),
        ],
    )(xp, wcat, bcat)


def _agg_h(acc_ref, z_ref):
    a = acc_ref[0, :, :D_H] + acc_ref[1, :, :D_H]
    cnt = acc_ref[0, :, D_H:D_H + 1] + acc_ref[1, :, D_H:D_H + 1]
    inv = 1.0 / jnp.maximum(cnt, 1.0)
    return a * inv + z_ref[...]


def _mid_body(acc_ref, z_ref, w_ref, b_ref, g_ref, zo_ref):
    h = jnp.maximum(_agg_h(acc_ref, z_ref), 0.0)
    o = jnp.dot(h, w_ref[...], preferred_element_type=F32) + b_ref[...]
    _pack_g(o, pl.program_id(0), g_ref, zo_ref)


def _mid(acc, z, wcat, bcat):
    return pl.pallas_call(
        _mid_body,
        grid=(N_TAB // M_BLK,),
        in_specs=[
            pl.BlockSpec((NC, M_BLK, WID), lambda i: (0, i, 0)),
            pl.BlockSpec((M_BLK, D_H), lambda i: (i, 0)),
            pl.BlockSpec((D_H, 2 * D_H), lambda i: (0, 0)),
            pl.BlockSpec((1, 2 * D_H), lambda i: (0, 0)),
        ],
        out_specs=[
            pl.BlockSpec((M_BLK, WID), lambda i: (i, 0)),
            pl.BlockSpec((M_BLK, D_H), lambda i: (i, 0)),
        ],
        out_shape=[
            jax.ShapeDtypeStruct((N_TAB, WID), F32),
            jax.ShapeDtypeStruct((N_TAB, D_H), F32),
        ],
    )(acc, z, wcat, bcat)


def _fin_body(acc_ref, z_ref, h_ref):
    h_ref[:, :D_H] = _agg_h(acc_ref, z_ref)
    h_ref[:, D_H:] = jnp.zeros((M_BLK, WID - D_H), F32)


def _fin(acc, z):
    return pl.pallas_call(
        _fin_body,
        grid=(N_TAB // M_BLK,),
        in_specs=[
            pl.BlockSpec((NC, M_BLK, WID), lambda i: (0, i, 0)),
            pl.BlockSpec((M_BLK, D_H), lambda i: (i, 0)),
        ],
        out_specs=pl.BlockSpec((M_BLK, WID), lambda i: (i, 0)),
        out_shape=jax.ShapeDtypeStruct((N_TAB, WID), F32),
    )(acc, z)


def _pad_pairs(a, b):
    e = a.shape[0]
    rows = -(-e // (CHUNK * NW * 8)) * (NW * 8)
    pad = N_REAL + (jnp.arange(rows * CHUNK - e, dtype=I32) % 16)
    a1 = jnp.concatenate([a.astype(I32), pad])
    b1 = jnp.concatenate([b.astype(I32), pad])
    return a1.reshape(rows, CHUNK), b1, rows


def kernel(x, edge_index, edge_label_index,
           W1_l, W1_r, b1, W2_l, W2_r, b2, W3_l, W3_r, b3):
    nl = edge_label_index.shape[1]

    xp = jnp.zeros((N_TAB, x.shape[1]), F32).at[:x.shape[0]].set(x)
    src2d, dst1d, e_rows = _pad_pairs(edge_index[0], edge_index[1])
    s2d, d1d, l_rows = _pad_pairs(edge_label_index[0], edge_label_index[1])
    d2d = d1d.reshape(l_rows, CHUNK)

    zero_tab = jnp.zeros((N_TAB, WID), F32)

    def cat(wl, wr, b):
        wc = jnp.concatenate([wl, wr], axis=1)
        bc = jnp.concatenate([jnp.zeros((D_H,), F32), b]).reshape(1, 2 * D_H)
        return wc, bc

    w1c, b1c = cat(W1_l, W1_r, b1)
    w2c, b2c = cat(W2_l, W2_r, b2)
    w3c, b3c = cat(W3_l, W3_r, b3)

    agg = _make_agg(e_rows)
    pred = _make_pred(l_rows)

    g1, z1 = _dense(xp, w1c, b1c)
    acc1 = agg(g1, src2d, dst1d, zero_tab)
    g2, z2 = _mid(acc1, z1, w2c, b2c)
    acc2 = agg(g2, src2d, dst1d, zero_tab)
    g3, z3 = _mid(acc2, z2, w3c, b3c)
    acc3 = agg(g3, src2d, dst1d, zero_tab)
    h3 = _fin(acc3, z3)
    out = pred(h3, s2d, d2d)
    return out[:nl]


# speedup vs baseline: 9.8764x; 1.2013x over previous
import functools

import jax
import jax.numpy as jnp
from jax import lax
from jax.experimental import pallas as pl
from jax.experimental.pallas import tpu as pltpu
from jax.experimental.pallas import tpu_sc as plsc

F32 = jnp.float32
I32 = jnp.int32

NC = 2
NS = 16
NW = NC * NS
CHUNK = 128
WID = 128
D_H = 64
N_REAL = 10000
N_TAB = 10240
RPT = N_TAB // NS


def _mesh():
    return plsc.VectorSubcoreMesh(core_axis_name="c", subcore_axis_name="s")


_SC_PARAMS = pltpu.CompilerParams(needs_layout_passes=False)


def _make_agg(n_chunk_rows):
    cpw = n_chunk_rows // NW
    hi = cpw // 2

    def body(g_hbm, src_hbm, dst_hbm, zero_hbm, acc_out,
             sidx, dring, buf0, buf1, acc_sh, gsem, ssem, dsem):
        c = lax.axis_index("c")
        s = lax.axis_index("s")
        wid = c * NS + s
        r0 = s * RPT
        row0 = wid * cpw
        pltpu.sync_copy(zero_hbm.at[pl.ds(r0, RPT)], acc_sh.at[pl.ds(r0, RPT)])
        pltpu.sync_copy(src_hbm.at[pl.ds(row0, cpw)], sidx)
        plsc.subcore_barrier()
        bufs = (buf0, buf1)

        def g_start(j, buf):
            pltpu.async_copy(g_hbm.at[sidx.at[j]], buf, gsem)

        def g_wait(j, buf):
            pltpu.make_async_copy(g_hbm.at[sidx.at[j]], buf, gsem).wait()

        def d_start(j, u):
            pltpu.async_copy(dst_hbm.at[pl.ds((row0 + j) * CHUNK, CHUNK)],
                             dring.at[u], dsem)

        def d_wait(j, u):
            pltpu.make_async_copy(dst_hbm.at[pl.ds((row0 + j) * CHUNK, CHUNK)],
                                  dring.at[u], dsem).wait()

        def s_start(j, u, buf):
            pltpu.async_copy(buf, acc_sh.at[dring.at[u]], ssem, add=True)

        def s_wait(j, u, buf):
            pltpu.make_async_copy(buf, acc_sh.at[dring.at[u]], ssem).wait()

        for u in range(4):
            d_start(u, u)
        g_start(0, buf0)

        def io_body(io, carry):
            base = 4 * io
            for u in range(4):
                j = base + u
                g_wait(j, bufs[u % 2])
                d_wait(j, u)

                @pl.when(j > 0)
                def _(u=u, j=j):
                    s_wait(j - 1, (u - 1) % 4, bufs[(u - 1) % 2])

                @pl.when(jnp.logical_and(j > 0, j + 3 < cpw))
                def _(u=u, j=j):
                    d_start(j + 3, (u + 3) % 4)

                @pl.when(j + 1 < cpw)
                def _(u=u, j=j):
                    g_start(j + 1, bufs[(u + 1) % 2])

                s_start(j, u, bufs[u % 2])
            return carry

        lax.fori_loop(0, cpw // 4, io_body, 0)
        s_wait(cpw - 1, 3, bufs[1])
        plsc.subcore_barrier()
        pltpu.sync_copy(acc_sh.at[pl.ds(r0, RPT)],
                        acc_out.at[c, pl.ds(r0, RPT)])

    return pl.kernel(
        body,
        out_type=jax.ShapeDtypeStruct((NC, N_TAB, WID), F32),
        mesh=_mesh(),
        compiler_params=_SC_PARAMS,
        scratch_types=(
            pltpu.VMEM((cpw, CHUNK), I32),
            pltpu.VMEM((4, CHUNK), I32),
            pltpu.VMEM((CHUNK, WID), F32),
            pltpu.VMEM((CHUNK, WID), F32),
            pltpu.VMEM_SHARED((N_TAB, WID), F32),
            pltpu.SemaphoreType.DMA,
            pltpu.SemaphoreType.DMA,
            pltpu.SemaphoreType.DMA,
        ),
    )


def _make_pred(n_chunk_rows):
    cpw = n_chunk_rows // NW
    hi = cpw // 2

    def body(h_hbm, s_hbm, d_hbm, out_hbm,
             sidx, didx, sr0, dr0, sr1, dr1, ovec, pbuf, sem):
        c = lax.axis_index("c")
        s = lax.axis_index("s")
        wid = c * NS + s
        row0 = wid * cpw
        pltpu.sync_copy(s_hbm.at[pl.ds(row0 * CHUNK, cpw * CHUNK)], sidx)
        pltpu.sync_copy(d_hbm.at[pl.ds(row0 * CHUNK, cpw * CHUNK)], didx)
        lanes = jnp.arange(16, dtype=I32)

        def g_start(j, sbuf, dbuf):
            pltpu.async_copy(h_hbm.at[sidx.at[pl.ds(j * CHUNK, CHUNK)]],
                             sbuf, sem)
            pltpu.async_copy(h_hbm.at[didx.at[pl.ds(j * CHUNK, CHUNK)]],
                             dbuf, sem)

        def g_wait(j, sbuf, dbuf):
            pltpu.make_async_copy(h_hbm.at[sidx.at[pl.ds(j * CHUNK, CHUNK)]],
                                  sbuf, sem).wait()
            pltpu.make_async_copy(h_hbm.at[didx.at[pl.ds(j * CHUNK, CHUNK)]],
                                  dbuf, sem).wait()

        def compute(j, sbuf, dbuf):
            def grp(g, carry2):
                for r in range(16):
                    p = jnp.zeros((16,), F32)
                    rr = g * 16 + r
                    for k in range(D_H // 16):
                        sv = sbuf[rr, pl.ds(k * 16, 16)]
                        dv = dbuf[rr, pl.ds(k * 16, 16)]
                        p = p + sv * dv
                    plsc.store_scatter(pbuf, [r * 16 + lanes], p)
                acc = jnp.zeros((16,), F32)
                for l in range(16):
                    acc = acc + plsc.load_gather(pbuf, [lanes * 16 + l])
                plsc.store_scatter(ovec, [g * 16 + lanes], acc)
                return carry2

            lax.fori_loop(0, CHUNK // 16, grp, 0)
            pltpu.sync_copy(ovec,
                            out_hbm.at[pl.ds((row0 + j) * CHUNK, CHUNK)])

        g_start(0, sr0, dr0)

        def io_body(io, carry):
            j0 = 2 * io
            j1 = j0 + 1
            g_wait(j0, sr0, dr0)
            g_start(j1, sr1, dr1)
            compute(j0, sr0, dr0)
            g_wait(j1, sr1, dr1)

            @pl.when(io < hi - 1)
            def _():
                g_start(j0 + 2, sr0, dr0)

            compute(j1, sr1, dr1)
            return carry

        lax.fori_loop(0, hi, io_body, 0)

    return pl.kernel(
        body,
        out_type=jax.ShapeDtypeStruct((n_chunk_rows * CHUNK,), F32),
        mesh=_mesh(),
        compiler_params=_SC_PARAMS,
        scratch_types=(
            pltpu.VMEM((cpw * CHUNK,), I32),
            pltpu.VMEM((cpw * CHUNK,), I32),
            pltpu.VMEM((CHUNK, WID), F32),
            pltpu.VMEM((CHUNK, WID), F32),
            pltpu.VMEM((CHUNK, WID), F32),
            pltpu.VMEM((CHUNK, WID), F32),
            pltpu.VMEM((CHUNK,), F32),
            pltpu.VMEM((256,), F32),
            pltpu.SemaphoreType.DMA,
        ),
    )


M_BLK = 1280


def _pack_g(o, blk_i, g_ref, z_ref):
    rows = blk_i * M_BLK + lax.broadcasted_iota(I32, (M_BLK, 1), 0)
    mask = rows < N_REAL
    g_ref[:, :D_H] = jnp.where(mask, o[:, :D_H], 0.0)
    g_ref[:, D_H:D_H + 1] = jnp.where(mask, 1.0, 0.0)
    g_ref[:, D_H + 1:] = jnp.zeros((M_BLK, WID - D_H - 1), F32)
    z_ref[...] = o[:, D_H:]


def _dense_body(x_ref, w_ref, b_ref, g_ref, z_ref):
    o = jnp.dot(x_ref[...], w_ref[...], preferred_element_type=F32)
    o = o + b_ref[...]
    _pack_g(o, pl.program_id(0), g_ref, z_ref)


def _dense(xp, wcat, bcat):
    k = xp.shape[1]
    return pl.pallas_call(
        _dense_body,
        grid=(N_TAB // M_BLK,),
        in_specs=[
            pl.BlockSpec((M_BLK, k), lambda i: (i, 0)),
            pl.BlockSpec((k, 2 * D_H), lambda i: (0, 0)),
            pl.BlockSpec((1, 2 * D_H), lambda i: (0, 0)),
        ],
        out_specs=[
            pl.BlockSpec((M_BLK, WID), lambda i: (i, 0)),
            pl.BlockSpec((M_BLK, D_H), lambda i: (i, 0)),
        ],
        out_shape=[
            jax.ShapeDtypeStruct((N_TAB, WID), F32),
            jax.ShapeDtypeStruct((N_TAB, D_H), F32),
        ],
    )(xp, wcat, bcat)


def _agg_h(acc_ref, z_ref):
    a = acc_ref[0, :, :D_H] + acc_ref[1, :, :D_H]
    cnt = acc_ref[0, :, D_H:D_H + 1] + acc_ref[1, :, D_H:D_H + 1]
    inv = 1.0 / jnp.maximum(cnt, 1.0)
    return a * inv + z_ref[...]


def _mid_body(acc_ref, z_ref, w_ref, b_ref, g_ref, zo_ref):
    h = jnp.maximum(_agg_h(acc_ref, z_ref), 0.0)
    o = jnp.dot(h, w_ref[...], preferred_element_type=F32) + b_ref[...]
    _pack_g(o, pl.program_id(0), g_ref, zo_ref)


def _mid(acc, z, wcat, bcat):
    return pl.pallas_call(
        _mid_body,
        grid=(N_TAB // M_BLK,),
        in_specs=[
            pl.BlockSpec((NC, M_BLK, WID), lambda i: (0, i, 0)),
            pl.BlockSpec((M_BLK, D_H), lambda i: (i, 0)),
            pl.BlockSpec((D_H, 2 * D_H), lambda i: (0, 0)),
            pl.BlockSpec((1, 2 * D_H), lambda i: (0, 0)),
        ],
        out_specs=[
            pl.BlockSpec((M_BLK, WID), lambda i: (i, 0)),
            pl.BlockSpec((M_BLK, D_H), lambda i: (i, 0)),
        ],
        out_shape=[
            jax.ShapeDtypeStruct((N_TAB, WID), F32),
            jax.ShapeDtypeStruct((N_TAB, D_H), F32),
        ],
    )(acc, z, wcat, bcat)


def _fin_body(acc_ref, z_ref, h_ref):
    h_ref[:, :D_H] = _agg_h(acc_ref, z_ref)
    h_ref[:, D_H:] = jnp.zeros((M_BLK, WID - D_H), F32)


def _fin(acc, z):
    return pl.pallas_call(
        _fin_body,
        grid=(N_TAB // M_BLK,),
        in_specs=[
            pl.BlockSpec((NC, M_BLK, WID), lambda i: (0, i, 0)),
            pl.BlockSpec((M_BLK, D_H), lambda i: (i, 0)),
        ],
        out_specs=pl.BlockSpec((M_BLK, WID), lambda i: (i, 0)),
        out_shape=jax.ShapeDtypeStruct((N_TAB, WID), F32),
    )(acc, z)


def _pad_pairs(a, b, mult):
    e = a.shape[0]
    rows = -(-e // (CHUNK * mult)) * mult
    pad = N_REAL + (jnp.arange(rows * CHUNK - e, dtype=I32) % 16)
    a1 = jnp.concatenate([a.astype(I32), pad])
    b1 = jnp.concatenate([b.astype(I32), pad])
    return a1, b1, rows


def kernel(x, edge_index, edge_label_index,
           W1_l, W1_r, b1, W2_l, W2_r, b2, W3_l, W3_r, b3):
    nl = edge_label_index.shape[1]

    xp = jnp.zeros((N_TAB, x.shape[1]), F32).at[:x.shape[0]].set(x)
    src1d, dst1d, e_rows = _pad_pairs(edge_index[0], edge_index[1], NW * 8)
    s1d, d1d, l_rows = _pad_pairs(edge_label_index[0], edge_label_index[1],
                                  NW * 2)
    src2d = src1d.reshape(e_rows, CHUNK)

    zero_tab = jnp.zeros((N_TAB, WID), F32)

    def cat(wl, wr, b):
        wc = jnp.concatenate([wl, wr], axis=1)
        bc = jnp.concatenate([jnp.zeros((D_H,), F32), b]).reshape(1, 2 * D_H)
        return wc, bc

    w1c, b1c = cat(W1_l, W1_r, b1)
    w2c, b2c = cat(W2_l, W2_r, b2)
    w3c, b3c = cat(W3_l, W3_r, b3)

    agg = _make_agg(e_rows)
    pred = _make_pred(l_rows)

    g1, z1 = _dense(xp, w1c, b1c)
    acc1 = agg(g1, src2d, dst1d, zero_tab)
    g2, z2 = _mid(acc1, z1, w2c, b2c)
    acc2 = agg(g2, src2d, dst1d, zero_tab)
    g3, z3 = _mid(acc2, z2, w3c, b3c)
    acc3 = agg(g3, src2d, dst1d, zero_tab)
    h3 = _fin(acc3, z3)
    out = pred(h3, s1d, d1d)
    return out[:nl]


# speedup vs baseline: 11.1500x; 1.1290x over previous
import functools

import jax
import jax.numpy as jnp
from jax import lax
from jax.experimental import pallas as pl
from jax.experimental.pallas import tpu as pltpu
from jax.experimental.pallas import tpu_sc as plsc

F32 = jnp.float32
I32 = jnp.int32

NC = 2
NS = 16
NW = NC * NS
CHUNK = 128
WID = 128
D_H = 64
N_REAL = 10000
N_TAB = 10240
RPT = N_TAB // NS


def _mesh():
    return plsc.VectorSubcoreMesh(core_axis_name="c", subcore_axis_name="s")


_SC_PARAMS = pltpu.CompilerParams(needs_layout_passes=False)


ECH = 64


def _make_agg(n_chunk_rows):
    cpw = n_chunk_rows // NW

    def body(g_hbm, src_hbm, dst_hbm, zero_hbm, acc_out,
             sring, dring, b0, b1, b2, b3, acc_sh, gsem, ssem, dsem, isem):
        c = lax.axis_index("c")
        s = lax.axis_index("s")
        wid = c * NS + s
        r0 = s * RPT
        row0 = wid * cpw
        pltpu.sync_copy(zero_hbm.at[pl.ds(r0, RPT)], acc_sh.at[pl.ds(r0, RPT)])
        bufs = (b0, b1, b2, b3)

        def i_start(j, u):
            pltpu.async_copy(src_hbm.at[pl.ds((row0 + j) * ECH, ECH)],
                             sring.at[u], isem)

        def i_wait(j, u):
            pltpu.make_async_copy(src_hbm.at[pl.ds((row0 + j) * ECH, ECH)],
                                  sring.at[u], isem).wait()

        def g_start(j, u, buf):
            pltpu.async_copy(g_hbm.at[sring.at[u]], buf, gsem)

        def g_wait(j, u, buf):
            pltpu.make_async_copy(g_hbm.at[sring.at[u]], buf, gsem).wait()

        def d_start(j, u):
            pltpu.async_copy(dst_hbm.at[pl.ds((row0 + j) * ECH, ECH)],
                             dring.at[u], dsem)

        def d_wait(j, u):
            pltpu.make_async_copy(dst_hbm.at[pl.ds((row0 + j) * ECH, ECH)],
                                  dring.at[u], dsem).wait()

        def s_start(j, u, buf):
            pltpu.async_copy(buf, acc_sh.at[dring.at[u]], ssem, add=True)

        def s_wait(j, u, buf):
            pltpu.make_async_copy(buf, acc_sh.at[dring.at[u]], ssem).wait()

        plsc.subcore_barrier()
        for u in range(4):
            i_start(u, u)
            d_start(u, u)
        i_wait(0, 0)
        g_start(0, 0, b0)
        i_wait(1, 1)
        g_start(1, 1, b1)

        def io_body(io, carry):
            base = 4 * io
            for u in range(4):
                j = base + u
                u2 = (u + 2) % 4
                g_wait(j, u, bufs[u])

                @pl.when(j + 4 < cpw)
                def _(u=u, j=j):
                    i_start(j + 4, u)

                d_wait(j, u)
                s_start(j, u, bufs[u])

                @pl.when(j >= 2)
                def _(u=u, j=j, u2=u2):
                    s_wait(j - 2, u2, bufs[u2])

                    @pl.when(j + 2 < cpw)
                    def _(u=u, j=j, u2=u2):
                        d_start(j + 2, u2)

                @pl.when(j + 2 < cpw)
                def _(u=u, j=j, u2=u2):
                    i_wait(j + 2, u2)
                    g_start(j + 2, u2, bufs[u2])
            return carry

        lax.fori_loop(0, cpw // 4, io_body, 0)
        s_wait(cpw - 2, 2, b2)
        s_wait(cpw - 1, 3, b3)
        plsc.subcore_barrier()
        pltpu.sync_copy(acc_sh.at[pl.ds(r0, RPT)],
                        acc_out.at[c, pl.ds(r0, RPT)])

    return pl.kernel(
        body,
        out_type=jax.ShapeDtypeStruct((NC, N_TAB, WID), F32),
        mesh=_mesh(),
        compiler_params=_SC_PARAMS,
        scratch_types=(
            pltpu.VMEM((4, ECH), I32),
            pltpu.VMEM((4, ECH), I32),
            pltpu.VMEM((ECH, WID), F32),
            pltpu.VMEM((ECH, WID), F32),
            pltpu.VMEM((ECH, WID), F32),
            pltpu.VMEM((ECH, WID), F32),
            pltpu.VMEM_SHARED((N_TAB, WID), F32),
            pltpu.SemaphoreType.DMA,
            pltpu.SemaphoreType.DMA,
            pltpu.SemaphoreType.DMA,
            pltpu.SemaphoreType.DMA,
        ),
    )


def _make_pred(n_chunk_rows):
    cpw = n_chunk_rows // NW
    hi = cpw // 2

    def body(h_hbm, s_hbm, d_hbm, out_hbm,
             sidx, didx, sr0, dr0, sr1, dr1, ovec, pbuf, sem):
        c = lax.axis_index("c")
        s = lax.axis_index("s")
        wid = c * NS + s
        row0 = wid * cpw
        pltpu.sync_copy(s_hbm.at[pl.ds(row0 * CHUNK, cpw * CHUNK)], sidx)
        pltpu.sync_copy(d_hbm.at[pl.ds(row0 * CHUNK, cpw * CHUNK)], didx)
        lanes = jnp.arange(16, dtype=I32)

        def g_start(j, sbuf, dbuf):
            pltpu.async_copy(h_hbm.at[sidx.at[pl.ds(j * CHUNK, CHUNK)]],
                             sbuf, sem)
            pltpu.async_copy(h_hbm.at[didx.at[pl.ds(j * CHUNK, CHUNK)]],
                             dbuf, sem)

        def g_wait(j, sbuf, dbuf):
            pltpu.make_async_copy(h_hbm.at[sidx.at[pl.ds(j * CHUNK, CHUNK)]],
                                  sbuf, sem).wait()
            pltpu.make_async_copy(h_hbm.at[didx.at[pl.ds(j * CHUNK, CHUNK)]],
                                  dbuf, sem).wait()

        def compute(j, sbuf, dbuf):
            def grp(g, carry2):
                for r in range(16):
                    p = jnp.zeros((16,), F32)
                    rr = g * 16 + r
                    for k in range(D_H // 16):
                        sv = sbuf[rr, pl.ds(k * 16, 16)]
                        dv = dbuf[rr, pl.ds(k * 16, 16)]
                        p = p + sv * dv
                    plsc.store_scatter(pbuf, [r * 16 + lanes], p)
                acc = jnp.zeros((16,), F32)
                for l in range(16):
                    acc = acc + plsc.load_gather(pbuf, [lanes * 16 + l])
                plsc.store_scatter(ovec, [g * 16 + lanes], acc)
                return carry2

            lax.fori_loop(0, CHUNK // 16, grp, 0)
            pltpu.sync_copy(ovec,
                            out_hbm.at[pl.ds((row0 + j) * CHUNK, CHUNK)])

        g_start(0, sr0, dr0)

        def io_body(io, carry):
            j0 = 2 * io
            j1 = j0 + 1
            g_wait(j0, sr0, dr0)
            g_start(j1, sr1, dr1)
            compute(j0, sr0, dr0)
            g_wait(j1, sr1, dr1)

            @pl.when(io < hi - 1)
            def _():
                g_start(j0 + 2, sr0, dr0)

            compute(j1, sr1, dr1)
            return carry

        lax.fori_loop(0, hi, io_body, 0)

    return pl.kernel(
        body,
        out_type=jax.ShapeDtypeStruct((n_chunk_rows * CHUNK,), F32),
        mesh=_mesh(),
        compiler_params=_SC_PARAMS,
        scratch_types=(
            pltpu.VMEM((cpw * CHUNK,), I32),
            pltpu.VMEM((cpw * CHUNK,), I32),
            pltpu.VMEM((CHUNK, WID), F32),
            pltpu.VMEM((CHUNK, WID), F32),
            pltpu.VMEM((CHUNK, WID), F32),
            pltpu.VMEM((CHUNK, WID), F32),
            pltpu.VMEM((CHUNK,), F32),
            pltpu.VMEM((256,), F32),
            pltpu.SemaphoreType.DMA,
        ),
    )


M_BLK = 1280


def _pack_g(o, blk_i, g_ref, z_ref):
    rows = blk_i * M_BLK + lax.broadcasted_iota(I32, (M_BLK, 1), 0)
    mask = rows < N_REAL
    g_ref[:, :D_H] = jnp.where(mask, o[:, :D_H], 0.0)
    g_ref[:, D_H:D_H + 1] = jnp.where(mask, 1.0, 0.0)
    g_ref[:, D_H + 1:] = jnp.zeros((M_BLK, WID - D_H - 1), F32)
    z_ref[...] = o[:, D_H:]


def _dense_body(x_ref, w_ref, b_ref, g_ref, z_ref):
    o = jnp.dot(x_ref[...], w_ref[...], preferred_element_type=F32)
    o = o + b_ref[...]
    _pack_g(o, pl.program_id(0), g_ref, z_ref)


def _dense(xp, wcat, bcat):
    k = xp.shape[1]
    return pl.pallas_call(
        _dense_body,
        grid=(N_TAB // M_BLK,),
        in_specs=[
            pl.BlockSpec((M_BLK, k), lambda i: (i, 0)),
            pl.BlockSpec((k, 2 * D_H), lambda i: (0, 0)),
            pl.BlockSpec((1, 2 * D_H), lambda i: (0, 0)),
        ],
        out_specs=[
            pl.BlockSpec((M_BLK, WID), lambda i: (i, 0)),
            pl.BlockSpec((M_BLK, D_H), lambda i: (i, 0)),
        ],
        out_shape=[
            jax.ShapeDtypeStruct((N_TAB, WID), F32),
            jax.ShapeDtypeStruct((N_TAB, D_H), F32),
        ],
    )(xp, wcat, bcat)


def _agg_h(acc_ref, z_ref):
    a = acc_ref[0, :, :D_H] + acc_ref[1, :, :D_H]
    cnt = acc_ref[0, :, D_H:D_H + 1] + acc_ref[1, :, D_H:D_H + 1]
    inv = 1.0 / jnp.maximum(cnt, 1.0)
    return a * inv + z_ref[...]


def _mid_body(acc_ref, z_ref, w_ref, b_ref, g_ref, zo_ref):
    h = jnp.maximum(_agg_h(acc_ref, z_ref), 0.0)
    o = jnp.dot(h, w_ref[...], preferred_element_type=F32) + b_ref[...]
    _pack_g(o, pl.program_id(0), g_ref, zo_ref)


def _mid(acc, z, wcat, bcat):
    return pl.pallas_call(
        _mid_body,
        grid=(N_TAB // M_BLK,),
        in_specs=[
            pl.BlockSpec((NC, M_BLK, WID), lambda i: (0, i, 0)),
            pl.BlockSpec((M_BLK, D_H), lambda i: (i, 0)),
            pl.BlockSpec((D_H, 2 * D_H), lambda i: (0, 0)),
            pl.BlockSpec((1, 2 * D_H), lambda i: (0, 0)),
        ],
        out_specs=[
            pl.BlockSpec((M_BLK, WID), lambda i: (i, 0)),
            pl.BlockSpec((M_BLK, D_H), lambda i: (i, 0)),
        ],
        out_shape=[
            jax.ShapeDtypeStruct((N_TAB, WID), F32),
            jax.ShapeDtypeStruct((N_TAB, D_H), F32),
        ],
    )(acc, z, wcat, bcat)


def _fin_body(acc_ref, z_ref, h_ref):
    h_ref[:, :D_H] = _agg_h(acc_ref, z_ref)
    h_ref[:, D_H:] = jnp.zeros((M_BLK, WID - D_H), F32)


def _fin(acc, z):
    return pl.pallas_call(
        _fin_body,
        grid=(N_TAB // M_BLK,),
        in_specs=[
            pl.BlockSpec((NC, M_BLK, WID), lambda i: (0, i, 0)),
            pl.BlockSpec((M_BLK, D_H), lambda i: (i, 0)),
        ],
        out_specs=pl.BlockSpec((M_BLK, WID), lambda i: (i, 0)),
        out_shape=jax.ShapeDtypeStruct((N_TAB, WID), F32),
    )(acc, z)


def _pad_pairs(a, b, mult):
    e = a.shape[0]
    rows = -(-e // (CHUNK * mult)) * mult
    pad = N_REAL + (jnp.arange(rows * CHUNK - e, dtype=I32) % 16)
    a1 = jnp.concatenate([a.astype(I32), pad])
    b1 = jnp.concatenate([b.astype(I32), pad])
    return a1, b1, rows


def kernel(x, edge_index, edge_label_index,
           W1_l, W1_r, b1, W2_l, W2_r, b2, W3_l, W3_r, b3):
    nl = edge_label_index.shape[1]

    xp = jnp.zeros((N_TAB, x.shape[1]), F32).at[:x.shape[0]].set(x)
    src1d, dst1d, e_rows = _pad_pairs(edge_index[0], edge_index[1], NW * 8)
    s1d, d1d, l_rows = _pad_pairs(edge_label_index[0], edge_label_index[1],
                                  NW * 2)

    zero_tab = jnp.zeros((N_TAB, WID), F32)

    def cat(wl, wr, b):
        wc = jnp.concatenate([wl, wr], axis=1)
        bc = jnp.concatenate([jnp.zeros((D_H,), F32), b]).reshape(1, 2 * D_H)
        return wc, bc

    w1c, b1c = cat(W1_l, W1_r, b1)
    w2c, b2c = cat(W2_l, W2_r, b2)
    w3c, b3c = cat(W3_l, W3_r, b3)

    agg = _make_agg(e_rows * (CHUNK // ECH))
    pred = _make_pred(l_rows)

    g1, z1 = _dense(xp, w1c, b1c)
    acc1 = agg(g1, src1d, dst1d, zero_tab)
    g2, z2 = _mid(acc1, z1, w2c, b2c)
    acc2 = agg(g2, src1d, dst1d, zero_tab)
    g3, z3 = _mid(acc2, z2, w3c, b3c)
    acc3 = agg(g3, src1d, dst1d, zero_tab)
    h3 = _fin(acc3, z3)
    out = pred(h3, s1d, d1d)
    return out[:nl]


# speedup vs baseline: 11.5168x; 1.0329x over previous
import functools

import jax
import jax.numpy as jnp
from jax import lax
from jax.experimental import pallas as pl
from jax.experimental.pallas import tpu as pltpu
from jax.experimental.pallas import tpu_sc as plsc

F32 = jnp.float32
I32 = jnp.int32

NC = 2
NS = 16
NW = NC * NS
CHUNK = 128
WID = 128
D_H = 64
N_REAL = 10000
N_TAB = 10240
RPT = N_TAB // NS


def _mesh():
    return plsc.VectorSubcoreMesh(core_axis_name="c", subcore_axis_name="s")


_SC_PARAMS = pltpu.CompilerParams(needs_layout_passes=False)


ECH = 80


def _make_agg(n_chunk_rows):
    cpw = n_chunk_rows // NW

    def body(g_hbm, src_hbm, dst_hbm, zero_hbm, acc_out,
             sring, dring, b0, b1, b2, b3, acc_sh, gsem, ssem, dsem, isem):
        c = lax.axis_index("c")
        s = lax.axis_index("s")
        wid = c * NS + s
        r0 = s * RPT
        row0 = wid * cpw
        pltpu.sync_copy(zero_hbm.at[pl.ds(r0, RPT)], acc_sh.at[pl.ds(r0, RPT)])
        bufs = (b0, b1, b2, b3)

        def i_start(j, u):
            pltpu.async_copy(src_hbm.at[pl.ds((row0 + j) * ECH, ECH)],
                             sring.at[u], isem)

        def i_wait(j, u):
            pltpu.make_async_copy(src_hbm.at[pl.ds((row0 + j) * ECH, ECH)],
                                  sring.at[u], isem).wait()

        def g_start(j, u, buf):
            pltpu.async_copy(g_hbm.at[sring.at[u]], buf, gsem)

        def g_wait(j, u, buf):
            pltpu.make_async_copy(g_hbm.at[sring.at[u]], buf, gsem).wait()

        def d_start(j, u):
            pltpu.async_copy(dst_hbm.at[pl.ds((row0 + j) * ECH, ECH)],
                             dring.at[u], dsem)

        def d_wait(j, u):
            pltpu.make_async_copy(dst_hbm.at[pl.ds((row0 + j) * ECH, ECH)],
                                  dring.at[u], dsem).wait()

        def s_start(j, u, buf):
            pltpu.async_copy(buf, acc_sh.at[dring.at[u]], ssem, add=True)

        def s_wait(j, u, buf):
            pltpu.make_async_copy(buf, acc_sh.at[dring.at[u]], ssem).wait()

        plsc.subcore_barrier()
        for u in range(4):
            i_start(u, u)
            d_start(u, u)
        i_wait(0, 0)
        g_start(0, 0, b0)
        i_wait(1, 1)
        g_start(1, 1, b1)

        def io_body(io, carry):
            base = 4 * io
            for u in range(4):
                j = base + u
                u2 = (u + 2) % 4
                g_wait(j, u, bufs[u])

                @pl.when(j + 4 < cpw)
                def _(u=u, j=j):
                    i_start(j + 4, u)

                d_wait(j, u)
                s_start(j, u, bufs[u])

                @pl.when(j >= 2)
                def _(u=u, j=j, u2=u2):
                    s_wait(j - 2, u2, bufs[u2])

                    @pl.when(j + 2 < cpw)
                    def _(u=u, j=j, u2=u2):
                        d_start(j + 2, u2)

                @pl.when(j + 2 < cpw)
                def _(u=u, j=j, u2=u2):
                    i_wait(j + 2, u2)
                    g_start(j + 2, u2, bufs[u2])
            return carry

        lax.fori_loop(0, cpw // 4, io_body, 0)
        s_wait(cpw - 2, 2, b2)
        s_wait(cpw - 1, 3, b3)
        plsc.subcore_barrier()
        pltpu.sync_copy(acc_sh.at[pl.ds(r0, RPT)],
                        acc_out.at[c, pl.ds(r0, RPT)])

    return pl.kernel(
        body,
        out_type=jax.ShapeDtypeStruct((NC, N_TAB, WID), F32),
        mesh=_mesh(),
        compiler_params=_SC_PARAMS,
        scratch_types=(
            pltpu.VMEM((4, ECH), I32),
            pltpu.VMEM((4, ECH), I32),
            pltpu.VMEM((ECH, WID), F32),
            pltpu.VMEM((ECH, WID), F32),
            pltpu.VMEM((ECH, WID), F32),
            pltpu.VMEM((ECH, WID), F32),
            pltpu.VMEM_SHARED((N_TAB, WID), F32),
            pltpu.SemaphoreType.DMA,
            pltpu.SemaphoreType.DMA,
            pltpu.SemaphoreType.DMA,
            pltpu.SemaphoreType.DMA,
        ),
    )


def _make_pred(n_chunk_rows):
    cpw = n_chunk_rows // NW
    hi = cpw // 2

    def body(h_hbm, s_hbm, d_hbm, out_hbm,
             sidx, didx, sr0, dr0, sr1, dr1, ovec, pbuf, sem):
        c = lax.axis_index("c")
        s = lax.axis_index("s")
        wid = c * NS + s
        row0 = wid * cpw
        pltpu.sync_copy(s_hbm.at[pl.ds(row0 * CHUNK, cpw * CHUNK)], sidx)
        pltpu.sync_copy(d_hbm.at[pl.ds(row0 * CHUNK, cpw * CHUNK)], didx)
        lanes = jnp.arange(16, dtype=I32)

        def g_start(j, sbuf, dbuf):
            pltpu.async_copy(h_hbm.at[sidx.at[pl.ds(j * CHUNK, CHUNK)]],
                             sbuf, sem)
            pltpu.async_copy(h_hbm.at[didx.at[pl.ds(j * CHUNK, CHUNK)]],
                             dbuf, sem)

        def g_wait(j, sbuf, dbuf):
            pltpu.make_async_copy(h_hbm.at[sidx.at[pl.ds(j * CHUNK, CHUNK)]],
                                  sbuf, sem).wait()
            pltpu.make_async_copy(h_hbm.at[didx.at[pl.ds(j * CHUNK, CHUNK)]],
                                  dbuf, sem).wait()

        def compute(j, sbuf, dbuf):
            def grp(g, carry2):
                for r in range(16):
                    p = jnp.zeros((16,), F32)
                    rr = g * 16 + r
                    for k in range(D_H // 16):
                        sv = sbuf[rr, pl.ds(k * 16, 16)]
                        dv = dbuf[rr, pl.ds(k * 16, 16)]
                        p = p + sv * dv
                    plsc.store_scatter(pbuf, [r * 16 + lanes], p)
                acc = jnp.zeros((16,), F32)
                for l in range(16):
                    acc = acc + plsc.load_gather(pbuf, [lanes * 16 + l])
                plsc.store_scatter(ovec, [g * 16 + lanes], acc)
                return carry2

            lax.fori_loop(0, CHUNK // 16, grp, 0)
            pltpu.sync_copy(ovec,
                            out_hbm.at[pl.ds((row0 + j) * CHUNK, CHUNK)])

        g_start(0, sr0, dr0)

        def io_body(io, carry):
            j0 = 2 * io
            j1 = j0 + 1
            g_wait(j0, sr0, dr0)
            g_start(j1, sr1, dr1)
            compute(j0, sr0, dr0)
            g_wait(j1, sr1, dr1)

            @pl.when(io < hi - 1)
            def _():
                g_start(j0 + 2, sr0, dr0)

            compute(j1, sr1, dr1)
            return carry

        lax.fori_loop(0, hi, io_body, 0)

    return pl.kernel(
        body,
        out_type=jax.ShapeDtypeStruct((n_chunk_rows * CHUNK,), F32),
        mesh=_mesh(),
        compiler_params=_SC_PARAMS,
        scratch_types=(
            pltpu.VMEM((cpw * CHUNK,), I32),
            pltpu.VMEM((cpw * CHUNK,), I32),
            pltpu.VMEM((CHUNK, WID), F32),
            pltpu.VMEM((CHUNK, WID), F32),
            pltpu.VMEM((CHUNK, WID), F32),
            pltpu.VMEM((CHUNK, WID), F32),
            pltpu.VMEM((CHUNK,), F32),
            pltpu.VMEM((256,), F32),
            pltpu.SemaphoreType.DMA,
        ),
    )


M_BLK = 1280


def _pack_g(o, blk_i, g_ref, z_ref):
    rows = blk_i * M_BLK + lax.broadcasted_iota(I32, (M_BLK, 1), 0)
    mask = rows < N_REAL
    g_ref[:, :D_H] = jnp.where(mask, o[:, :D_H], 0.0)
    g_ref[:, D_H:D_H + 1] = jnp.where(mask, 1.0, 0.0)
    g_ref[:, D_H + 1:] = jnp.zeros((M_BLK, WID - D_H - 1), F32)
    z_ref[...] = o[:, D_H:]


def _dense_body(x_ref, w_ref, b_ref, g_ref, z_ref):
    o = jnp.dot(x_ref[...], w_ref[...], preferred_element_type=F32)
    o = o + b_ref[...]
    _pack_g(o, pl.program_id(0), g_ref, z_ref)


def _dense(xp, wcat, bcat):
    k = xp.shape[1]
    return pl.pallas_call(
        _dense_body,
        grid=(N_TAB // M_BLK,),
        in_specs=[
            pl.BlockSpec((M_BLK, k), lambda i: (i, 0)),
            pl.BlockSpec((k, 2 * D_H), lambda i: (0, 0)),
            pl.BlockSpec((1, 2 * D_H), lambda i: (0, 0)),
        ],
        out_specs=[
            pl.BlockSpec((M_BLK, WID), lambda i: (i, 0)),
            pl.BlockSpec((M_BLK, D_H), lambda i: (i, 0)),
        ],
        out_shape=[
            jax.ShapeDtypeStruct((N_TAB, WID), F32),
            jax.ShapeDtypeStruct((N_TAB, D_H), F32),
        ],
    )(xp, wcat, bcat)


def _agg_h(acc_ref, z_ref):
    a = acc_ref[0, :, :D_H] + acc_ref[1, :, :D_H]
    cnt = acc_ref[0, :, D_H:D_H + 1] + acc_ref[1, :, D_H:D_H + 1]
    inv = 1.0 / jnp.maximum(cnt, 1.0)
    return a * inv + z_ref[...]


def _mid_body(acc_ref, z_ref, w_ref, b_ref, g_ref, zo_ref):
    h = jnp.maximum(_agg_h(acc_ref, z_ref), 0.0)
    o = jnp.dot(h, w_ref[...], preferred_element_type=F32) + b_ref[...]
    _pack_g(o, pl.program_id(0), g_ref, zo_ref)


def _mid(acc, z, wcat, bcat):
    return pl.pallas_call(
        _mid_body,
        grid=(N_TAB // M_BLK,),
        in_specs=[
            pl.BlockSpec((NC, M_BLK, WID), lambda i: (0, i, 0)),
            pl.BlockSpec((M_BLK, D_H), lambda i: (i, 0)),
            pl.BlockSpec((D_H, 2 * D_H), lambda i: (0, 0)),
            pl.BlockSpec((1, 2 * D_H), lambda i: (0, 0)),
        ],
        out_specs=[
            pl.BlockSpec((M_BLK, WID), lambda i: (i, 0)),
            pl.BlockSpec((M_BLK, D_H), lambda i: (i, 0)),
        ],
        out_shape=[
            jax.ShapeDtypeStruct((N_TAB, WID), F32),
            jax.ShapeDtypeStruct((N_TAB, D_H), F32),
        ],
    )(acc, z, wcat, bcat)


def _fin_body(acc_ref, z_ref, h_ref):
    h_ref[:, :D_H] = _agg_h(acc_ref, z_ref)
    h_ref[:, D_H:] = jnp.zeros((M_BLK, WID - D_H), F32)


def _fin(acc, z):
    return pl.pallas_call(
        _fin_body,
        grid=(N_TAB // M_BLK,),
        in_specs=[
            pl.BlockSpec((NC, M_BLK, WID), lambda i: (0, i, 0)),
            pl.BlockSpec((M_BLK, D_H), lambda i: (i, 0)),
        ],
        out_specs=pl.BlockSpec((M_BLK, WID), lambda i: (i, 0)),
        out_shape=jax.ShapeDtypeStruct((N_TAB, WID), F32),
    )(acc, z)


def _pad_pairs(a, b, mult):
    e = a.shape[0]
    rows = -(-e // (CHUNK * mult)) * mult
    pad = N_REAL + (jnp.arange(rows * CHUNK - e, dtype=I32) % 16)
    a1 = jnp.concatenate([a.astype(I32), pad])
    b1 = jnp.concatenate([b.astype(I32), pad])
    return a1, b1, rows


def kernel(x, edge_index, edge_label_index,
           W1_l, W1_r, b1, W2_l, W2_r, b2, W3_l, W3_r, b3):
    nl = edge_label_index.shape[1]

    xp = jnp.zeros((N_TAB, x.shape[1]), F32).at[:x.shape[0]].set(x)
    src1d, dst1d, e_rows = _pad_pairs(edge_index[0], edge_index[1], NW * 8)
    s1d, d1d, l_rows = _pad_pairs(edge_label_index[0], edge_label_index[1],
                                  NW * 2)

    zero_tab = jnp.zeros((N_TAB, WID), F32)

    def cat(wl, wr, b):
        wc = jnp.concatenate([wl, wr], axis=1)
        bc = jnp.concatenate([jnp.zeros((D_H,), F32), b]).reshape(1, 2 * D_H)
        return wc, bc

    w1c, b1c = cat(W1_l, W1_r, b1)
    w2c, b2c = cat(W2_l, W2_r, b2)
    w3c, b3c = cat(W3_l, W3_r, b3)

    agg = _make_agg(e_rows * CHUNK // ECH)
    pred = _make_pred(l_rows)

    g1, z1 = _dense(xp, w1c, b1c)
    acc1 = agg(g1, src1d, dst1d, zero_tab)
    g2, z2 = _mid(acc1, z1, w2c, b2c)
    acc2 = agg(g2, src1d, dst1d, zero_tab)
    g3, z3 = _mid(acc2, z2, w3c, b3c)
    acc3 = agg(g3, src1d, dst1d, zero_tab)
    h3 = _fin(acc3, z3)
    out = pred(h3, s1d, d1d)
    return out[:nl]


# speedup vs baseline: 11.5438x; 1.0023x over previous
import functools

import jax
import jax.numpy as jnp
from jax import lax
from jax.experimental import pallas as pl
from jax.experimental.pallas import tpu as pltpu
from jax.experimental.pallas import tpu_sc as plsc

F32 = jnp.float32
I32 = jnp.int32

NC = 2
NS = 16
NW = NC * NS
CHUNK = 128
WID = 128
D_H = 64
N_REAL = 10000
N_TAB = 10240
RPT = N_TAB // NS


def _mesh():
    return plsc.VectorSubcoreMesh(core_axis_name="c", subcore_axis_name="s")


_SC_PARAMS = pltpu.CompilerParams(needs_layout_passes=False)


ECH = 80


def _make_agg(n_chunk_rows):
    cpw = n_chunk_rows // NW

    def body(g_hbm, src_hbm, dst_hbm, zero_hbm, acc_out,
             sring, dring, b0, b1, b2, b3, acc_sh, gsem, ssem, dsem, isem):
        c = lax.axis_index("c")
        s = lax.axis_index("s")
        wid = c * NS + s
        r0 = s * RPT
        row0 = wid * cpw
        pltpu.sync_copy(zero_hbm.at[pl.ds(r0, RPT)], acc_sh.at[pl.ds(r0, RPT)])
        bufs = (b0, b1, b2, b3)

        def i_start(j, u):
            pltpu.async_copy(src_hbm.at[pl.ds((row0 + j) * ECH, ECH)],
                             sring.at[u], isem)

        def i_wait(j, u):
            pltpu.make_async_copy(src_hbm.at[pl.ds((row0 + j) * ECH, ECH)],
                                  sring.at[u], isem).wait()

        def g_start(j, u, buf):
            pltpu.async_copy(g_hbm.at[sring.at[u]], buf, gsem)

        def g_wait(j, u, buf):
            pltpu.make_async_copy(g_hbm.at[sring.at[u]], buf, gsem).wait()

        def d_start(j, u):
            pltpu.async_copy(dst_hbm.at[pl.ds((row0 + j) * ECH, ECH)],
                             dring.at[u], dsem)

        def d_wait(j, u):
            pltpu.make_async_copy(dst_hbm.at[pl.ds((row0 + j) * ECH, ECH)],
                                  dring.at[u], dsem).wait()

        def s_start(j, u, buf):
            pltpu.async_copy(buf, acc_sh.at[dring.at[u]], ssem, add=True)

        def s_wait(j, u, buf):
            pltpu.make_async_copy(buf, acc_sh.at[dring.at[u]], ssem).wait()

        plsc.subcore_barrier()
        for u in range(4):
            i_start(u, u)
            d_start(u, u)
        i_wait(0, 0)
        g_start(0, 0, b0)
        i_wait(1, 1)
        g_start(1, 1, b1)

        def io_body(io, carry):
            base = 4 * io
            for u in range(4):
                j = base + u
                u2 = (u + 2) % 4
                g_wait(j, u, bufs[u])

                @pl.when(j + 4 < cpw)
                def _(u=u, j=j):
                    i_start(j + 4, u)

                d_wait(j, u)
                s_start(j, u, bufs[u])

                @pl.when(j >= 2)
                def _(u=u, j=j, u2=u2):
                    s_wait(j - 2, u2, bufs[u2])

                    @pl.when(j + 2 < cpw)
                    def _(u=u, j=j, u2=u2):
                        d_start(j + 2, u2)

                @pl.when(j + 2 < cpw)
                def _(u=u, j=j, u2=u2):
                    i_wait(j + 2, u2)
                    g_start(j + 2, u2, bufs[u2])
            return carry

        lax.fori_loop(0, cpw // 4, io_body, 0)
        s_wait(cpw - 2, 2, b2)
        s_wait(cpw - 1, 3, b3)
        plsc.subcore_barrier()
        pltpu.sync_copy(acc_sh.at[pl.ds(r0, RPT)],
                        acc_out.at[c, pl.ds(r0, RPT)])

    return pl.kernel(
        body,
        out_type=jax.ShapeDtypeStruct((NC, N_TAB, WID), F32),
        mesh=_mesh(),
        compiler_params=_SC_PARAMS,
        scratch_types=(
            pltpu.VMEM((4, ECH), I32),
            pltpu.VMEM((4, ECH), I32),
            pltpu.VMEM((ECH, WID), F32),
            pltpu.VMEM((ECH, WID), F32),
            pltpu.VMEM((ECH, WID), F32),
            pltpu.VMEM((ECH, WID), F32),
            pltpu.VMEM_SHARED((N_TAB, WID), F32),
            pltpu.SemaphoreType.DMA,
            pltpu.SemaphoreType.DMA,
            pltpu.SemaphoreType.DMA,
            pltpu.SemaphoreType.DMA,
        ),
    )


def _make_pred(n_chunk_rows):
    cpw = n_chunk_rows // NW
    hi = cpw // 2

    def body(h_hbm, s_hbm, d_hbm, out_hbm,
             sidx, didx, sr0, dr0, sr1, dr1, ovec0, ovec1, pbuf, sem, osem):
        c = lax.axis_index("c")
        s = lax.axis_index("s")
        wid = c * NS + s
        row0 = wid * cpw
        pltpu.sync_copy(s_hbm.at[pl.ds(row0 * CHUNK, cpw * CHUNK)], sidx)
        pltpu.sync_copy(d_hbm.at[pl.ds(row0 * CHUNK, cpw * CHUNK)], didx)
        lanes = jnp.arange(16, dtype=I32)

        def g_start(j, sbuf, dbuf):
            pltpu.async_copy(h_hbm.at[sidx.at[pl.ds(j * CHUNK, CHUNK)]],
                             sbuf, sem)
            pltpu.async_copy(h_hbm.at[didx.at[pl.ds(j * CHUNK, CHUNK)]],
                             dbuf, sem)

        def g_wait(j, sbuf, dbuf):
            pltpu.make_async_copy(h_hbm.at[sidx.at[pl.ds(j * CHUNK, CHUNK)]],
                                  sbuf, sem).wait()
            pltpu.make_async_copy(h_hbm.at[didx.at[pl.ds(j * CHUNK, CHUNK)]],
                                  dbuf, sem).wait()

        def compute(j, sbuf, dbuf, ovec):
            def grp(g, carry2):
                for r in range(16):
                    p = jnp.zeros((16,), F32)
                    rr = g * 16 + r
                    for k in range(D_H // 16):
                        sv = sbuf[rr, pl.ds(k * 16, 16)]
                        dv = dbuf[rr, pl.ds(k * 16, 16)]
                        p = p + sv * dv
                    plsc.store_scatter(pbuf, [r * 16 + lanes], p)
                acc = jnp.zeros((16,), F32)
                for l in range(16):
                    acc = acc + plsc.load_gather(pbuf, [lanes * 16 + l])
                plsc.store_scatter(ovec, [g * 16 + lanes], acc)
                return carry2

            lax.fori_loop(0, CHUNK // 16, grp, 0)
            pltpu.async_copy(ovec,
                             out_hbm.at[pl.ds((row0 + j) * CHUNK, CHUNK)],
                             osem)

        def o_wait(j, ovec):
            pltpu.make_async_copy(ovec,
                                  out_hbm.at[pl.ds((row0 + j) * CHUNK, CHUNK)],
                                  osem).wait()

        g_start(0, sr0, dr0)

        def io_body(io, carry):
            j0 = 2 * io
            j1 = j0 + 1
            g_wait(j0, sr0, dr0)
            g_start(j1, sr1, dr1)

            @pl.when(io > 0)
            def _():
                o_wait(j0 - 2, ovec0)

            compute(j0, sr0, dr0, ovec0)
            g_wait(j1, sr1, dr1)

            @pl.when(io < hi - 1)
            def _():
                g_start(j0 + 2, sr0, dr0)

            @pl.when(io > 0)
            def _():
                o_wait(j0 - 1, ovec1)

            compute(j1, sr1, dr1, ovec1)
            return carry

        lax.fori_loop(0, hi, io_body, 0)
        o_wait(cpw - 2, ovec0)
        o_wait(cpw - 1, ovec1)

    return pl.kernel(
        body,
        out_type=jax.ShapeDtypeStruct((n_chunk_rows * CHUNK,), F32),
        mesh=_mesh(),
        compiler_params=_SC_PARAMS,
        scratch_types=(
            pltpu.VMEM((cpw * CHUNK,), I32),
            pltpu.VMEM((cpw * CHUNK,), I32),
            pltpu.VMEM((CHUNK, WID), F32),
            pltpu.VMEM((CHUNK, WID), F32),
            pltpu.VMEM((CHUNK, WID), F32),
            pltpu.VMEM((CHUNK, WID), F32),
            pltpu.VMEM((CHUNK,), F32),
            pltpu.VMEM((CHUNK,), F32),
            pltpu.VMEM((256,), F32),
            pltpu.SemaphoreType.DMA,
            pltpu.SemaphoreType.DMA,
        ),
    )


M_BLK = 1280


def _pack_g(o, blk_i, g_ref, z_ref):
    rows = blk_i * M_BLK + lax.broadcasted_iota(I32, (M_BLK, 1), 0)
    mask = rows < N_REAL
    g_ref[:, :D_H] = jnp.where(mask, o[:, :D_H], 0.0)
    g_ref[:, D_H:D_H + 1] = jnp.where(mask, 1.0, 0.0)
    g_ref[:, D_H + 1:] = jnp.zeros((M_BLK, WID - D_H - 1), F32)
    z_ref[...] = o[:, D_H:]


def _dense_body(x_ref, w_ref, b_ref, g_ref, z_ref):
    o = jnp.dot(x_ref[...], w_ref[...], preferred_element_type=F32)
    o = o + b_ref[...]
    _pack_g(o, pl.program_id(0), g_ref, z_ref)


def _dense(xp, wcat, bcat):
    k = xp.shape[1]
    return pl.pallas_call(
        _dense_body,
        grid=(N_TAB // M_BLK,),
        in_specs=[
            pl.BlockSpec((M_BLK, k), lambda i: (i, 0)),
            pl.BlockSpec((k, 2 * D_H), lambda i: (0, 0)),
            pl.BlockSpec((1, 2 * D_H), lambda i: (0, 0)),
        ],
        out_specs=[
            pl.BlockSpec((M_BLK, WID), lambda i: (i, 0)),
            pl.BlockSpec((M_BLK, D_H), lambda i: (i, 0)),
        ],
        out_shape=[
            jax.ShapeDtypeStruct((N_TAB, WID), F32),
            jax.ShapeDtypeStruct((N_TAB, D_H), F32),
        ],
    )(xp, wcat, bcat)


def _agg_h(acc_ref, z_ref):
    a = acc_ref[0, :, :D_H] + acc_ref[1, :, :D_H]
    cnt = acc_ref[0, :, D_H:D_H + 1] + acc_ref[1, :, D_H:D_H + 1]
    inv = 1.0 / jnp.maximum(cnt, 1.0)
    return a * inv + z_ref[...]


def _mid_body(acc_ref, z_ref, w_ref, b_ref, g_ref, zo_ref):
    h = jnp.maximum(_agg_h(acc_ref, z_ref), 0.0)
    o = jnp.dot(h, w_ref[...], preferred_element_type=F32) + b_ref[...]
    _pack_g(o, pl.program_id(0), g_ref, zo_ref)


def _mid(acc, z, wcat, bcat):
    return pl.pallas_call(
        _mid_body,
        grid=(N_TAB // M_BLK,),
        in_specs=[
            pl.BlockSpec((NC, M_BLK, WID), lambda i: (0, i, 0)),
            pl.BlockSpec((M_BLK, D_H), lambda i: (i, 0)),
            pl.BlockSpec((D_H, 2 * D_H), lambda i: (0, 0)),
            pl.BlockSpec((1, 2 * D_H), lambda i: (0, 0)),
        ],
        out_specs=[
            pl.BlockSpec((M_BLK, WID), lambda i: (i, 0)),
            pl.BlockSpec((M_BLK, D_H), lambda i: (i, 0)),
        ],
        out_shape=[
            jax.ShapeDtypeStruct((N_TAB, WID), F32),
            jax.ShapeDtypeStruct((N_TAB, D_H), F32),
        ],
    )(acc, z, wcat, bcat)


def _fin_body(acc_ref, z_ref, h_ref):
    h_ref[:, :D_H] = _agg_h(acc_ref, z_ref)
    h_ref[:, D_H:] = jnp.zeros((M_BLK, WID - D_H), F32)


def _fin(acc, z):
    return pl.pallas_call(
        _fin_body,
        grid=(N_TAB // M_BLK,),
        in_specs=[
            pl.BlockSpec((NC, M_BLK, WID), lambda i: (0, i, 0)),
            pl.BlockSpec((M_BLK, D_H), lambda i: (i, 0)),
        ],
        out_specs=pl.BlockSpec((M_BLK, WID), lambda i: (i, 0)),
        out_shape=jax.ShapeDtypeStruct((N_TAB, WID), F32),
    )(acc, z)


def _pad_pairs(a, b, mult):
    e = a.shape[0]
    rows = -(-e // (CHUNK * mult)) * mult
    pad = N_REAL + (jnp.arange(rows * CHUNK - e, dtype=I32) % 16)
    a1 = jnp.concatenate([a.astype(I32), pad])
    b1 = jnp.concatenate([b.astype(I32), pad])
    return a1, b1, rows


def kernel(x, edge_index, edge_label_index,
           W1_l, W1_r, b1, W2_l, W2_r, b2, W3_l, W3_r, b3):
    nl = edge_label_index.shape[1]

    xp = jnp.zeros((N_TAB, x.shape[1]), F32).at[:x.shape[0]].set(x)
    src1d, dst1d, e_rows = _pad_pairs(edge_index[0], edge_index[1], NW * 8)
    s1d, d1d, l_rows = _pad_pairs(edge_label_index[0], edge_label_index[1],
                                  NW * 2)

    zero_tab = jnp.zeros((N_TAB, WID), F32)

    def cat(wl, wr, b):
        wc = jnp.concatenate([wl, wr], axis=1)
        bc = jnp.concatenate([jnp.zeros((D_H,), F32), b]).reshape(1, 2 * D_H)
        return wc, bc

    w1c, b1c = cat(W1_l, W1_r, b1)
    w2c, b2c = cat(W2_l, W2_r, b2)
    w3c, b3c = cat(W3_l, W3_r, b3)

    agg = _make_agg(e_rows * CHUNK // ECH)
    pred = _make_pred(l_rows)

    g1, z1 = _dense(xp, w1c, b1c)
    acc1 = agg(g1, src1d, dst1d, zero_tab)
    g2, z2 = _mid(acc1, z1, w2c, b2c)
    acc2 = agg(g2, src1d, dst1d, zero_tab)
    g3, z3 = _mid(acc2, z2, w3c, b3c)
    acc3 = agg(g3, src1d, dst1d, zero_tab)
    h3 = _fin(acc3, z3)
    out = pred(h3, s1d, d1d)
    return out[:nl]


# speedup vs baseline: 11.5553x; 1.0010x over previous
import jax
import jax.numpy as jnp
from jax import lax
from jax.experimental import pallas as pl
from jax.experimental.pallas import tpu as pltpu
from jax.experimental.pallas import tpu_sc as plsc

F32 = jnp.float32
I32 = jnp.int32

NC = 2
NS = 16
NW = NC * NS
CHUNK = 128
WID = 128
D_H = 64
N_REAL = 10000
N_TAB = 10240
RPT = N_TAB // NS


def _mesh():
    return plsc.VectorSubcoreMesh(core_axis_name="c", subcore_axis_name="s")


_SC_PARAMS = pltpu.CompilerParams(needs_layout_passes=False)


ECH = 80


def _make_agg(n_chunk_rows):
    cpw = n_chunk_rows // NW

    def body(g_hbm, src_hbm, dst_hbm, zero_hbm, acc_out,
             sring, dring, b0, b1, b2, b3, acc_sh, gsem, ssem, dsem, isem):
        c = lax.axis_index("c")
        s = lax.axis_index("s")
        wid = c * NS + s
        r0 = s * RPT
        row0 = wid * cpw
        pltpu.sync_copy(zero_hbm.at[pl.ds(r0, RPT)], acc_sh.at[pl.ds(r0, RPT)])
        bufs = (b0, b1, b2, b3)

        def i_start(j, u):
            pltpu.async_copy(src_hbm.at[pl.ds((row0 + j) * ECH, ECH)],
                             sring.at[u], isem)

        def i_wait(j, u):
            pltpu.make_async_copy(src_hbm.at[pl.ds((row0 + j) * ECH, ECH)],
                                  sring.at[u], isem).wait()

        def g_start(j, u, buf):
            pltpu.async_copy(g_hbm.at[sring.at[u]], buf, gsem)

        def g_wait(j, u, buf):
            pltpu.make_async_copy(g_hbm.at[sring.at[u]], buf, gsem).wait()

        def d_start(j, u):
            pltpu.async_copy(dst_hbm.at[pl.ds((row0 + j) * ECH, ECH)],
                             dring.at[u], dsem)

        def d_wait(j, u):
            pltpu.make_async_copy(dst_hbm.at[pl.ds((row0 + j) * ECH, ECH)],
                                  dring.at[u], dsem).wait()

        def s_start(j, u, buf):
            pltpu.async_copy(buf, acc_sh.at[dring.at[u]], ssem, add=True)

        def s_wait(j, u, buf):
            pltpu.make_async_copy(buf, acc_sh.at[dring.at[u]], ssem).wait()

        plsc.subcore_barrier()
        for u in range(4):
            i_start(u, u)
            d_start(u, u)
        i_wait(0, 0)
        g_start(0, 0, b0)
        i_wait(1, 1)
        g_start(1, 1, b1)

        def io_body(io, carry):
            base = 4 * io
            for u in range(4):
                j = base + u
                u2 = (u + 2) % 4
                g_wait(j, u, bufs[u])

                @pl.when(j + 4 < cpw)
                def _(u=u, j=j):
                    i_start(j + 4, u)

                d_wait(j, u)
                s_start(j, u, bufs[u])

                @pl.when(j >= 2)
                def _(u=u, j=j, u2=u2):
                    s_wait(j - 2, u2, bufs[u2])

                    @pl.when(j + 2 < cpw)
                    def _(u=u, j=j, u2=u2):
                        d_start(j + 2, u2)

                @pl.when(j + 2 < cpw)
                def _(u=u, j=j, u2=u2):
                    i_wait(j + 2, u2)
                    g_start(j + 2, u2, bufs[u2])
            return carry

        lax.fori_loop(0, cpw // 4, io_body, 0)
        s_wait(cpw - 2, 2, b2)
        s_wait(cpw - 1, 3, b3)
        plsc.subcore_barrier()
        pltpu.sync_copy(acc_sh.at[pl.ds(r0, RPT)],
                        acc_out.at[c, pl.ds(r0, RPT)])

    return pl.kernel(
        body,
        out_type=jax.ShapeDtypeStruct((NC, N_TAB, WID), F32),
        mesh=_mesh(),
        compiler_params=_SC_PARAMS,
        scratch_types=(
            pltpu.VMEM((4, ECH), I32),
            pltpu.VMEM((4, ECH), I32),
            pltpu.VMEM((ECH, WID), F32),
            pltpu.VMEM((ECH, WID), F32),
            pltpu.VMEM((ECH, WID), F32),
            pltpu.VMEM((ECH, WID), F32),
            pltpu.VMEM_SHARED((N_TAB, WID), F32),
            pltpu.SemaphoreType.DMA,
            pltpu.SemaphoreType.DMA,
            pltpu.SemaphoreType.DMA,
            pltpu.SemaphoreType.DMA,
        ),
    )


def _make_pred(n_chunk_rows):
    cpw = n_chunk_rows // NW
    hi = cpw // 2

    def body(h_hbm, s_hbm, d_hbm, out_hbm,
             sidx, didx, sr0, dr0, sr1, dr1, ovec0, ovec1, pbuf, sem, osem):
        c = lax.axis_index("c")
        s = lax.axis_index("s")
        wid = c * NS + s
        row0 = wid * cpw
        pltpu.sync_copy(s_hbm.at[pl.ds(row0 * CHUNK, cpw * CHUNK)], sidx)
        pltpu.sync_copy(d_hbm.at[pl.ds(row0 * CHUNK, cpw * CHUNK)], didx)
        lanes = jnp.arange(16, dtype=I32)

        def g_start(j, sbuf, dbuf):
            pltpu.async_copy(h_hbm.at[sidx.at[pl.ds(j * CHUNK, CHUNK)]],
                             sbuf, sem)
            pltpu.async_copy(h_hbm.at[didx.at[pl.ds(j * CHUNK, CHUNK)]],
                             dbuf, sem)

        def g_wait(j, sbuf, dbuf):
            pltpu.make_async_copy(h_hbm.at[sidx.at[pl.ds(j * CHUNK, CHUNK)]],
                                  sbuf, sem).wait()
            pltpu.make_async_copy(h_hbm.at[didx.at[pl.ds(j * CHUNK, CHUNK)]],
                                  dbuf, sem).wait()

        def compute(j, sbuf, dbuf, ovec):
            def grp(g, carry2):
                for r in range(16):
                    p = jnp.zeros((16,), F32)
                    rr = g * 16 + r
                    for k in range(D_H // 16):
                        sv = sbuf[rr, pl.ds(k * 16, 16)]
                        dv = dbuf[rr, pl.ds(k * 16, 16)]
                        p = p + sv * dv
                    plsc.store_scatter(pbuf, [r * 16 + lanes], p)
                acc = jnp.zeros((16,), F32)
                for l in range(16):
                    acc = acc + plsc.load_gather(pbuf, [lanes * 16 + l])
                plsc.store_scatter(ovec, [g * 16 + lanes], acc)
                return carry2

            lax.fori_loop(0, CHUNK // 16, grp, 0)
            pltpu.async_copy(ovec,
                             out_hbm.at[pl.ds((row0 + j) * CHUNK, CHUNK)],
                             osem)

        def o_wait(j, ovec):
            pltpu.make_async_copy(ovec,
                                  out_hbm.at[pl.ds((row0 + j) * CHUNK, CHUNK)],
                                  osem).wait()

        g_start(0, sr0, dr0)

        def io_body(io, carry):
            j0 = 2 * io
            j1 = j0 + 1
            g_wait(j0, sr0, dr0)
            g_start(j1, sr1, dr1)

            @pl.when(io > 0)
            def _():
                o_wait(j0 - 2, ovec0)

            compute(j0, sr0, dr0, ovec0)
            g_wait(j1, sr1, dr1)

            @pl.when(io < hi - 1)
            def _():
                g_start(j0 + 2, sr0, dr0)

            @pl.when(io > 0)
            def _():
                o_wait(j0 - 1, ovec1)

            compute(j1, sr1, dr1, ovec1)
            return carry

        lax.fori_loop(0, hi, io_body, 0)
        o_wait(cpw - 2, ovec0)
        o_wait(cpw - 1, ovec1)

    return pl.kernel(
        body,
        out_type=jax.ShapeDtypeStruct((n_chunk_rows * CHUNK,), F32),
        mesh=_mesh(),
        compiler_params=_SC_PARAMS,
        scratch_types=(
            pltpu.VMEM((cpw * CHUNK,), I32),
            pltpu.VMEM((cpw * CHUNK,), I32),
            pltpu.VMEM((CHUNK, WID), F32),
            pltpu.VMEM((CHUNK, WID), F32),
            pltpu.VMEM((CHUNK, WID), F32),
            pltpu.VMEM((CHUNK, WID), F32),
            pltpu.VMEM((CHUNK,), F32),
            pltpu.VMEM((CHUNK,), F32),
            pltpu.VMEM((256,), F32),
            pltpu.SemaphoreType.DMA,
            pltpu.SemaphoreType.DMA,
        ),
    )


M_BLK = 1280


def _pack_g(o, blk_i, g_ref, z_ref):
    rows = blk_i * M_BLK + lax.broadcasted_iota(I32, (M_BLK, 1), 0)
    mask = rows < N_REAL
    g_ref[:, :D_H] = jnp.where(mask, o[:, :D_H], 0.0)
    g_ref[:, D_H:D_H + 1] = jnp.where(mask, 1.0, 0.0)
    g_ref[:, D_H + 1:] = jnp.zeros((M_BLK, WID - D_H - 1), F32)
    z_ref[...] = o[:, D_H:]


def _dense_body(x_ref, w_ref, b_ref, g_ref, z_ref):
    o = jnp.dot(x_ref[...], w_ref[...], preferred_element_type=F32)
    o = o + b_ref[...]
    _pack_g(o, pl.program_id(0), g_ref, z_ref)


def _dense(xp, wcat, bcat):
    k = xp.shape[1]
    return pl.pallas_call(
        _dense_body,
        grid=(N_TAB // M_BLK,),
        in_specs=[
            pl.BlockSpec((M_BLK, k), lambda i: (i, 0)),
            pl.BlockSpec((k, 2 * D_H), lambda i: (0, 0)),
            pl.BlockSpec((1, 2 * D_H), lambda i: (0, 0)),
        ],
        out_specs=[
            pl.BlockSpec((M_BLK, WID), lambda i: (i, 0)),
            pl.BlockSpec((M_BLK, D_H), lambda i: (i, 0)),
        ],
        out_shape=[
            jax.ShapeDtypeStruct((N_TAB, WID), F32),
            jax.ShapeDtypeStruct((N_TAB, D_H), F32),
        ],
    )(xp, wcat, bcat)


def _agg_h(acc_ref, z_ref):
    a = acc_ref[0, :, :D_H] + acc_ref[1, :, :D_H]
    cnt = acc_ref[0, :, D_H:D_H + 1] + acc_ref[1, :, D_H:D_H + 1]
    inv = 1.0 / jnp.maximum(cnt, 1.0)
    return a * inv + z_ref[...]


def _mid_body(acc_ref, z_ref, w_ref, b_ref, g_ref, zo_ref):
    h = jnp.maximum(_agg_h(acc_ref, z_ref), 0.0)
    o = jnp.dot(h, w_ref[...], preferred_element_type=F32) + b_ref[...]
    _pack_g(o, pl.program_id(0), g_ref, zo_ref)


def _mid(acc, z, wcat, bcat):
    return pl.pallas_call(
        _mid_body,
        grid=(N_TAB // M_BLK,),
        in_specs=[
            pl.BlockSpec((NC, M_BLK, WID), lambda i: (0, i, 0)),
            pl.BlockSpec((M_BLK, D_H), lambda i: (i, 0)),
            pl.BlockSpec((D_H, 2 * D_H), lambda i: (0, 0)),
            pl.BlockSpec((1, 2 * D_H), lambda i: (0, 0)),
        ],
        out_specs=[
            pl.BlockSpec((M_BLK, WID), lambda i: (i, 0)),
            pl.BlockSpec((M_BLK, D_H), lambda i: (i, 0)),
        ],
        out_shape=[
            jax.ShapeDtypeStruct((N_TAB, WID), F32),
            jax.ShapeDtypeStruct((N_TAB, D_H), F32),
        ],
    )(acc, z, wcat, bcat)


def _fin_body(acc_ref, z_ref, h_ref):
    h_ref[:, :D_H] = _agg_h(acc_ref, z_ref)
    h_ref[:, D_H:] = jnp.zeros((M_BLK, WID - D_H), F32)


def _fin(acc, z):
    return pl.pallas_call(
        _fin_body,
        grid=(N_TAB // M_BLK,),
        in_specs=[
            pl.BlockSpec((NC, M_BLK, WID), lambda i: (0, i, 0)),
            pl.BlockSpec((M_BLK, D_H), lambda i: (i, 0)),
        ],
        out_specs=pl.BlockSpec((M_BLK, WID), lambda i: (i, 0)),
        out_shape=jax.ShapeDtypeStruct((N_TAB, WID), F32),
    )(acc, z)


def _pad_pairs(a, b, mult):
    e = a.shape[0]
    rows = -(-e // (CHUNK * mult)) * mult
    pad = N_REAL + (jnp.arange(rows * CHUNK - e, dtype=I32) % 16)
    a1 = jnp.concatenate([a.astype(I32), pad])
    b1 = jnp.concatenate([b.astype(I32), pad])
    return a1, b1, rows


def kernel(x, edge_index, edge_label_index,
           W1_l, W1_r, b1, W2_l, W2_r, b2, W3_l, W3_r, b3):
    nl = edge_label_index.shape[1]

    xp = jnp.zeros((N_TAB, x.shape[1]), F32).at[:x.shape[0]].set(x)
    src1d, dst1d, e_rows = _pad_pairs(edge_index[0], edge_index[1], NW * 8)
    s1d, d1d, l_rows = _pad_pairs(edge_label_index[0], edge_label_index[1],
                                  NW * 2)

    zero_tab = jnp.zeros((N_TAB, WID), F32)

    def cat(wl, wr, b):
        wc = jnp.concatenate([wl, wr], axis=1)
        bc = jnp.concatenate([jnp.zeros((D_H,), F32), b]).reshape(1, 2 * D_H)
        return wc, bc

    w1c, b1c = cat(W1_l, W1_r, b1)
    w2c, b2c = cat(W2_l, W2_r, b2)
    w3c, b3c = cat(W3_l, W3_r, b3)

    agg = _make_agg(e_rows * CHUNK // ECH)
    pred = _make_pred(l_rows)

    g1, z1 = _dense(xp, w1c, b1c)
    acc1 = agg(g1, src1d, dst1d, zero_tab)
    g2, z2 = _mid(acc1, z1, w2c, b2c)
    acc2 = agg(g2, src1d, dst1d, zero_tab)
    g3, z3 = _mid(acc2, z2, w3c, b3c)
    acc3 = agg(g3, src1d, dst1d, zero_tab)
    h3 = _fin(acc3, z3)
    out = pred(h3, s1d, d1d)
    return out[:nl]
